# Initial kernel scaffold; baseline (speedup 1.0000x reference)
import jax
import jax.numpy as jnp
from jax.experimental import pallas as pl


def kernel(inputs, W_router, W_gate, W_up, W_down):
    raise NotImplementedError("write your pallas kernel here")


# speedup vs baseline: 7.4866x; 7.4866x over previous
import functools

import jax
import jax.numpy as jnp
from jax.experimental import pallas as pl
from jax.experimental.pallas import tpu as pltpu

_E = 16
_D = 768
_DFF = 2048
_T = 2048
_BM = 128
_NTILES = 32
_TPAD = _NTILES * _BM


def _router_body(x_ref, wr_ref, out_ref):
    out_ref[...] = jax.lax.dot_general(
        x_ref[...], wr_ref[...], (((1,), (1,)), ((), ())),
        preferred_element_type=jnp.float32)


def _router_logits(x, w_router):
    return pl.pallas_call(
        _router_body,
        out_shape=jax.ShapeDtypeStruct((_T, _E), jnp.float32),
    )(x, w_router)


def _gemm_body(eot_ref, valid_ref, x_ref, wg_ref, wu_ref, wd_ref, y_ref):
    s = pl.program_id(0)

    @pl.when(valid_ref[s] > 0)
    def _():
        x = x_ref[...]
        g = jax.lax.dot_general(x, wg_ref[0], (((1,), (1,)), ((), ())),
                                preferred_element_type=jnp.float32)
        u = jax.lax.dot_general(x, wu_ref[0], (((1,), (1,)), ((), ())),
                                preferred_element_type=jnp.float32)
        h = g * jax.nn.sigmoid(g) * u
        y_ref[...] = jax.lax.dot_general(h, wd_ref[0], (((1,), (1,)), ((), ())),
                                         preferred_element_type=jnp.float32)


def _grouped_gemm(eot, valid, x_padded, w_gate, w_up, w_down):
    grid_spec = pltpu.PrefetchScalarGridSpec(
        num_scalar_prefetch=2,
        grid=(_NTILES,),
        in_specs=[
            pl.BlockSpec((_BM, _D), lambda s, eot, valid: (s, 0)),
            pl.BlockSpec((1, _DFF, _D), lambda s, eot, valid: (eot[s], 0, 0)),
            pl.BlockSpec((1, _DFF, _D), lambda s, eot, valid: (eot[s], 0, 0)),
            pl.BlockSpec((1, _D, _DFF), lambda s, eot, valid: (eot[s], 0, 0)),
        ],
        out_specs=pl.BlockSpec((_BM, _D), lambda s, eot, valid: (s, 0)),
    )
    return pl.pallas_call(
        _gemm_body,
        grid_spec=grid_spec,
        out_shape=jax.ShapeDtypeStruct((_TPAD, _D), jnp.float32),
    )(eot, valid, x_padded, w_gate, w_up, w_down)


def kernel(inputs, W_router, W_gate, W_up, W_down):
    x = inputs
    logits = _router_logits(x, W_router)

    lmax = jnp.max(logits, axis=-1)
    w_tok = 1.0 / jnp.sum(jnp.exp(logits - lmax[:, None]), axis=-1)
    eid = jnp.argmax(logits, axis=-1).astype(jnp.int32)
    onehot = jax.nn.one_hot(eid, _E, dtype=jnp.int32)
    counts = jnp.sum(onehot, axis=0)
    rank = jnp.take_along_axis(jnp.cumsum(onehot, axis=0), eid[:, None], 1)[:, 0] - 1
    padded = ((counts + _BM - 1) // _BM) * _BM
    base = jnp.cumsum(padded) - padded
    pos = base[eid] + rank

    x_padded = jnp.zeros((_TPAD, _D), jnp.float32).at[pos].set(x)

    s_idx = jnp.arange(_NTILES)
    tile_start = base // _BM
    tile_end = (base + padded) // _BM
    in_range = (s_idx[:, None] >= tile_start[None]) & (s_idx[:, None] < tile_end[None])
    eot = jnp.sum(jnp.arange(_E)[None] * in_range, axis=1).astype(jnp.int32)
    valid = jnp.any(in_range, axis=1).astype(jnp.int32)
    eot = jnp.where(valid == 1, eot, jnp.max(eot * valid)).astype(jnp.int32)

    y_padded = _grouped_gemm(eot, valid, x_padded, W_gate, W_up, W_down)
    return y_padded[pos] * w_tok[:, None]
